# Initial kernel scaffold; baseline (speedup 1.0000x reference)
import jax
import jax.numpy as jnp
from jax.experimental import pallas as pl


def kernel(feat, edge_index, weight, bias):
    raise NotImplementedError("write your pallas kernel here")


# speedup vs baseline: 11.5754x; 11.5754x over previous
import dataclasses
import functools

import jax
import jax.numpy as jnp
from jax import lax
from jax.experimental import pallas as pl
from jax.experimental.pallas import tpu as pltpu
from jax.experimental.pallas import tpu_sc as plsc

NC = 2
NS = 16
NW = NC * NS
L = 16


def _sc_params():
    cp = pltpu.CompilerParams()
    if "needs_layout_passes" in pltpu.CompilerParams.__dataclass_fields__:
        cp = dataclasses.replace(cp, needs_layout_passes=False)
    return cp


def _sc_mesh():
    return plsc.VectorSubcoreMesh(
        core_axis_name="c", subcore_axis_name="s",
        num_cores=NC, num_subcores=NS)


def _pick_window(ew: int) -> int:
    for w in range(128, 0, -8):
        if ew % w == 0:
            return w
    return 8


def _sc_degree(dst_flat, n):
    ew = dst_flat.shape[1]

    @functools.partial(
        pl.kernel,
        mesh=_sc_mesh(),
        out_type=jax.ShapeDtypeStruct((NW, n), jnp.float32),
        compiler_params=_sc_params(),
        scratch_types=[
            pltpu.VMEM((ew,), jnp.int32),
            pltpu.VMEM((n,), jnp.float32),
        ],
    )
    def k(dst_hbm, out_hbm, idx_v, acc_v):
        wid = lax.axis_index("s") * NC + lax.axis_index("c")
        zero = jnp.zeros((L,), jnp.float32)

        @pl.loop(0, n, step=L)
        def _(i):
            acc_v[pl.ds(i, L)] = zero

        pltpu.sync_copy(dst_hbm.at[wid], idx_v)
        ones = jnp.ones((L,), jnp.float32)

        @pl.loop(0, ew, step=L)
        def _(i):
            idx = idx_v[pl.ds(i, L)]
            plsc.addupdate_scatter(acc_v, [idx], ones)

        pltpu.sync_copy(acc_v, out_hbm.at[wid])

    return k(dst_flat)


def _sc_hop(x, src_flat, dst_r, n, d):
    nwin, w = dst_r.shape[1], dst_r.shape[2]
    ch = (n // NS) & ~7
    tail = n - ch * NS
    assert tail % 8 == 0 and tail <= ch

    @functools.partial(
        pl.kernel,
        mesh=_sc_mesh(),
        out_type=jax.ShapeDtypeStruct((NC, n, d), jnp.float32),
        compiler_params=_sc_params(),
        scratch_types=[
            pltpu.VMEM((nwin * w,), jnp.int32),
            pltpu.VMEM((nwin, w), jnp.int32),
            pltpu.VMEM((w, d), jnp.float32),
            pltpu.VMEM((w, d), jnp.float32),
            pltpu.VMEM_SHARED((n, d), jnp.float32),
            pltpu.SemaphoreType.DMA,
            pltpu.SemaphoreType.DMA,
        ],
    )
    def k(x_hbm, src_hbm, dst_hbm, zero_hbm, out_hbm,
          src_v, dst_v, buf_a, buf_b, acc_sh, sem_a, sem_b):
        cid = lax.axis_index("c")
        sid = lax.axis_index("s")
        wid = sid * NC + cid

        pltpu.sync_copy(src_hbm.at[wid], src_v)
        pltpu.sync_copy(dst_hbm.at[wid], dst_v)

        pltpu.sync_copy(zero_hbm, acc_sh.at[pl.ds(sid * ch, ch)])

        @pl.when(sid == NS - 1)
        def _():
            pltpu.sync_copy(zero_hbm.at[pl.ds(0, tail)],
                            acc_sh.at[pl.ds(NS * ch, tail)])

        plsc.subcore_barrier()

        pltpu.async_copy(x_hbm.at[src_v.at[pl.ds(0, w)]], buf_a, sem_a)

        @pl.loop(0, nwin, step=2)
        def _(j):
            @pl.when(j + 1 < nwin)
            def _():
                pltpu.async_copy(
                    x_hbm.at[src_v.at[pl.ds((j + 1) * w, w)]], buf_b, sem_b)

            pltpu.make_async_copy(
                x_hbm.at[src_v.at[pl.ds(j * w, w)]], buf_a, sem_a).wait()
            pltpu.sync_copy(buf_a, acc_sh.at[dst_v.at[j]], add=True)

            @pl.when(j + 2 < nwin)
            def _():
                pltpu.async_copy(
                    x_hbm.at[src_v.at[pl.ds((j + 2) * w, w)]], buf_a, sem_a)

            @pl.when(j + 1 < nwin)
            def _():
                pltpu.make_async_copy(
                    x_hbm.at[src_v.at[pl.ds((j + 1) * w, w)]],
                    buf_b, sem_b).wait()
                pltpu.sync_copy(buf_b, acc_sh.at[dst_v.at[j + 1]], add=True)

        plsc.subcore_barrier()
        pltpu.sync_copy(acc_sh.at[pl.ds(sid * ch, ch)],
                        out_hbm.at[cid].at[pl.ds(sid * ch, ch)])

        @pl.when(sid == NS - 1)
        def _():
            pltpu.sync_copy(acc_sh.at[pl.ds(NS * ch, tail)],
                            out_hbm.at[cid].at[pl.ds(NS * ch, tail)])

    zeros = jnp.zeros((ch, d), jnp.float32)
    return k(x, src_flat, dst_r, zeros)


def _tc_scale(deg_parts, feat, n, d, mb):
    def body(dp_ref, f_ref, x1_ref):
        nrm = lax.rsqrt(jnp.sum(dp_ref[...], axis=1))[:, None]
        x1_ref[...] = f_ref[...] * nrm

    return pl.pallas_call(
        body,
        grid=(n // mb,),
        in_specs=[
            pl.BlockSpec((mb, NW), lambda i: (i, 0)),
            pl.BlockSpec((mb, d), lambda i: (i, 0)),
        ],
        out_specs=pl.BlockSpec((mb, d), lambda i: (i, 0)),
        out_shape=jax.ShapeDtypeStruct((n, d), jnp.float32),
    )(deg_parts, feat)


def _tc_mid(deg_parts, h_parts, n, d, mb):
    def body(dp_ref, hp_ref, rst_ref, xn_ref):
        nrm = lax.rsqrt(jnp.sum(dp_ref[...], axis=1))[:, None]
        r = (hp_ref[0] + hp_ref[1]) * nrm
        rst_ref[...] = r
        xn_ref[...] = r * nrm

    return pl.pallas_call(
        body,
        grid=(n // mb,),
        in_specs=[
            pl.BlockSpec((mb, NW), lambda i: (i, 0)),
            pl.BlockSpec((NC, mb, d), lambda i: (0, i, 0)),
        ],
        out_specs=[
            pl.BlockSpec((mb, d), lambda i: (i, 0)),
            pl.BlockSpec((mb, d), lambda i: (i, 0)),
        ],
        out_shape=[
            jax.ShapeDtypeStruct((n, d), jnp.float32),
            jax.ShapeDtypeStruct((n, d), jnp.float32),
        ],
    )(deg_parts, h_parts)


def _tc_final(deg_parts, feat, rst1, h2_parts, w_stk, bias2, n, d, out_d, mb):
    def body(dp_ref, f_ref, r1_ref, hp_ref, w_ref, b_ref, o_ref):
        nrm = lax.rsqrt(jnp.sum(dp_ref[...], axis=1))[:, None]
        r2 = (hp_ref[0] + hp_ref[1]) * nrm
        acc = jnp.dot(f_ref[...], w_ref[0],
                      preferred_element_type=jnp.float32)
        acc = acc + jnp.dot(r1_ref[...], w_ref[1],
                            preferred_element_type=jnp.float32)
        acc = acc + jnp.dot(r2, w_ref[2],
                            preferred_element_type=jnp.float32)
        o_ref[...] = acc + b_ref[...]

    return pl.pallas_call(
        body,
        grid=(n // mb,),
        in_specs=[
            pl.BlockSpec((mb, NW), lambda i: (i, 0)),
            pl.BlockSpec((mb, d), lambda i: (i, 0)),
            pl.BlockSpec((mb, d), lambda i: (i, 0)),
            pl.BlockSpec((NC, mb, d), lambda i: (0, i, 0)),
            pl.BlockSpec((3, d, out_d), lambda i: (0, 0, 0)),
            pl.BlockSpec((1, out_d), lambda i: (0, 0)),
        ],
        out_specs=pl.BlockSpec((mb, out_d), lambda i: (i, 0)),
        out_shape=jax.ShapeDtypeStruct((n, out_d), jnp.float32),
    )(deg_parts, feat, rst1, h2_parts, w_stk, bias2)


def kernel(feat, edge_index, weight, bias):
    n, d = feat.shape
    out_d = weight.shape[0]
    e = edge_index.shape[1]
    assert e % NW == 0 and n % NS == 0
    ew = e // NW
    w = _pick_window(ew)
    nwin = ew // w

    src_flat = edge_index[0].reshape(NW, ew)
    dst_r = edge_index[1].reshape(NW, nwin, w)
    dst_flat = edge_index[1].reshape(NW, ew)

    mb = 2000 if n % 2000 == 0 else n

    deg_parts = _sc_degree(dst_flat, n).T
    x1 = _tc_scale(deg_parts, feat, n, d, mb)
    h1_parts = _sc_hop(x1, src_flat, dst_r, n, d)
    rst1, x2 = _tc_mid(deg_parts, h1_parts, n, d, mb)
    h2_parts = _sc_hop(x2, src_flat, dst_r, n, d)

    w_stk = weight.T.reshape(3, d, out_d)
    bias2 = bias.reshape(1, out_d)
    return _tc_final(deg_parts, feat, rst1, h2_parts, w_stk, bias2,
                     n, d, out_d, mb)


# speedup vs baseline: 11.9701x; 1.0341x over previous
import dataclasses
import functools

import jax
import jax.numpy as jnp
from jax import lax
from jax.experimental import pallas as pl
from jax.experimental.pallas import tpu as pltpu
from jax.experimental.pallas import tpu_sc as plsc

NC = 2
NS = 16
NW = NC * NS
L = 16


def _sc_params():
    cp = pltpu.CompilerParams()
    if "needs_layout_passes" in pltpu.CompilerParams.__dataclass_fields__:
        cp = dataclasses.replace(cp, needs_layout_passes=False)
    return cp


def _sc_mesh():
    return plsc.VectorSubcoreMesh(
        core_axis_name="c", subcore_axis_name="s",
        num_cores=NC, num_subcores=NS)


def _pick_window(ew: int) -> int:
    for w in range(128, 0, -8):
        if ew % w == 0:
            return w
    return 8


def _sc_degree(dst_flat, n):
    ew = dst_flat.shape[1]

    @functools.partial(
        pl.kernel,
        mesh=_sc_mesh(),
        out_type=jax.ShapeDtypeStruct((NW, n), jnp.float32),
        compiler_params=_sc_params(),
        scratch_types=[
            pltpu.VMEM((ew,), jnp.int32),
            pltpu.VMEM((n,), jnp.float32),
        ],
    )
    def k(dst_hbm, out_hbm, idx_v, acc_v):
        wid = lax.axis_index("s") * NC + lax.axis_index("c")
        zero = jnp.zeros((L,), jnp.float32)

        @pl.loop(0, n, step=L)
        def _(i):
            acc_v[pl.ds(i, L)] = zero

        pltpu.sync_copy(dst_hbm.at[wid], idx_v)
        ones = jnp.ones((L,), jnp.float32)

        @pl.loop(0, ew, step=L)
        def _(i):
            idx = idx_v[pl.ds(i, L)]
            plsc.addupdate_scatter(acc_v, [idx], ones)

        pltpu.sync_copy(acc_v, out_hbm.at[wid])

    return k(dst_flat)


def _sc_hop(x, src_flat, dst_r, n, d):
    nwin, w = dst_r.shape[1], dst_r.shape[2]
    ch = (n // NS) & ~7
    tail = n - ch * NS
    assert tail % 8 == 0 and tail <= ch

    @functools.partial(
        pl.kernel,
        mesh=_sc_mesh(),
        out_type=jax.ShapeDtypeStruct((NC, n, d), jnp.float32),
        compiler_params=_sc_params(),
        scratch_types=[
            pltpu.VMEM((nwin * w,), jnp.int32),
            pltpu.VMEM((nwin, w), jnp.int32),
            pltpu.VMEM((w, d), jnp.float32),
            pltpu.VMEM((w, d), jnp.float32),
            pltpu.VMEM_SHARED((n, d), jnp.float32),
            pltpu.SemaphoreType.DMA,
            pltpu.SemaphoreType.DMA,
        ],
    )
    def k(x_hbm, src_hbm, dst_hbm, zero_hbm, out_hbm,
          src_v, dst_v, buf_a, buf_b, acc_sh, sem_a, sem_b):
        cid = lax.axis_index("c")
        sid = lax.axis_index("s")
        wid = sid * NC + cid

        pltpu.sync_copy(src_hbm.at[wid], src_v)
        pltpu.sync_copy(dst_hbm.at[wid], dst_v)

        pltpu.sync_copy(zero_hbm, acc_sh.at[pl.ds(sid * ch, ch)])

        @pl.when(sid == NS - 1)
        def _():
            pltpu.sync_copy(zero_hbm.at[pl.ds(0, tail)],
                            acc_sh.at[pl.ds(NS * ch, tail)])

        plsc.subcore_barrier()

        pltpu.async_copy(x_hbm.at[src_v.at[pl.ds(0, w)]], buf_a, sem_a)

        @pl.loop(0, nwin, step=2)
        def _(j):
            @pl.when(j + 1 < nwin)
            def _():
                pltpu.async_copy(
                    x_hbm.at[src_v.at[pl.ds((j + 1) * w, w)]], buf_b, sem_b)

            pltpu.make_async_copy(
                x_hbm.at[src_v.at[pl.ds(j * w, w)]], buf_a, sem_a).wait()
            pltpu.sync_copy(buf_a, acc_sh.at[dst_v.at[j]], add=True)

            @pl.when(j + 2 < nwin)
            def _():
                pltpu.async_copy(
                    x_hbm.at[src_v.at[pl.ds((j + 2) * w, w)]], buf_a, sem_a)

            @pl.when(j + 1 < nwin)
            def _():
                pltpu.make_async_copy(
                    x_hbm.at[src_v.at[pl.ds((j + 1) * w, w)]],
                    buf_b, sem_b).wait()
                pltpu.sync_copy(buf_b, acc_sh.at[dst_v.at[j + 1]], add=True)

        plsc.subcore_barrier()
        pltpu.sync_copy(acc_sh.at[pl.ds(sid * ch, ch)],
                        out_hbm.at[cid].at[pl.ds(sid * ch, ch)])

        @pl.when(sid == NS - 1)
        def _():
            pltpu.sync_copy(acc_sh.at[pl.ds(NS * ch, tail)],
                            out_hbm.at[cid].at[pl.ds(NS * ch, tail)])

    zeros = jnp.zeros((ch, d), jnp.float32)
    return k(x, src_flat, dst_r, zeros)


def _tc_scale(deg_parts, feat, n, d):
    def body(dp_ref, f_ref, x1_ref):
        nrm = lax.rsqrt(jnp.sum(dp_ref[...], axis=0))[:, None]
        x1_ref[...] = f_ref[...] * nrm

    return pl.pallas_call(
        body,
        out_shape=jax.ShapeDtypeStruct((n, d), jnp.float32),
    )(deg_parts, feat)


def _tc_mid(deg_parts, h_parts, n, d):
    def body(dp_ref, hp_ref, rst_ref, xn_ref):
        nrm = lax.rsqrt(jnp.sum(dp_ref[...], axis=0))[:, None]
        r = (hp_ref[0] + hp_ref[1]) * nrm
        rst_ref[...] = r
        xn_ref[...] = r * nrm

    return pl.pallas_call(
        body,
        out_shape=[
            jax.ShapeDtypeStruct((n, d), jnp.float32),
            jax.ShapeDtypeStruct((n, d), jnp.float32),
        ],
    )(deg_parts, h_parts)


def _tc_final(deg_parts, feat, rst1, h2_parts, w_stk, bias2, n, d, out_d):
    def body(dp_ref, f_ref, r1_ref, hp_ref, w_ref, b_ref, o_ref):
        nrm = lax.rsqrt(jnp.sum(dp_ref[...], axis=0))[:, None]
        r2 = (hp_ref[0] + hp_ref[1]) * nrm
        acc = jnp.dot(f_ref[...], w_ref[0],
                      preferred_element_type=jnp.float32)
        acc = acc + jnp.dot(r1_ref[...], w_ref[1],
                            preferred_element_type=jnp.float32)
        acc = acc + jnp.dot(r2, w_ref[2],
                            preferred_element_type=jnp.float32)
        o_ref[...] = acc + b_ref[...]

    return pl.pallas_call(
        body,
        out_shape=jax.ShapeDtypeStruct((n, out_d), jnp.float32),
    )(deg_parts, feat, rst1, h2_parts, w_stk, bias2)


def kernel(feat, edge_index, weight, bias):
    n, d = feat.shape
    out_d = weight.shape[0]
    e = edge_index.shape[1]
    assert e % NW == 0 and n % NS == 0
    ew = e // NW
    w = _pick_window(ew)
    nwin = ew // w

    src_flat = edge_index[0].reshape(NW, ew)
    dst_r = edge_index[1].reshape(NW, nwin, w)
    dst_flat = edge_index[1].reshape(NW, ew)

    deg_parts = _sc_degree(dst_flat, n)
    x1 = _tc_scale(deg_parts, feat, n, d)
    h1_parts = _sc_hop(x1, src_flat, dst_r, n, d)
    rst1, x2 = _tc_mid(deg_parts, h1_parts, n, d)
    h2_parts = _sc_hop(x2, src_flat, dst_r, n, d)

    w_stk = weight.T.reshape(3, d, out_d)
    bias2 = bias.reshape(1, out_d)
    return _tc_final(deg_parts, feat, rst1, h2_parts, w_stk, bias2,
                     n, d, out_d)
